# baseline (device time: 175766 ns/iter reference)
import jax
import jax.numpy as jnp
from jax import lax
from jax.experimental import pallas as pl
from jax.experimental.pallas import tpu as pltpu

N_DEV = 16


def kernel(x, w_mat, scale_x, scale_w):
    m_per, k = x.shape
    _, n = w_mat.shape
    n_per = n // N_DEV

    def body(x_ref, w_ref, sx_ref, sw_ref, out_ref, y_scratch, send_sem, recv_sem):
        jj = pl.program_id(0)
        my_i = lax.axis_index("i")
        tgt = lax.rem(my_i + jj, N_DEV)

        scale = sx_ref[0] * sw_ref[0]
        xb = x_ref[...].astype(jnp.bfloat16)
        wb = w_ref[...].astype(jnp.bfloat16)
        y = jnp.dot(xb, wb, preferred_element_type=jnp.float32) * scale

        @pl.when(jj == 0)
        def _():
            out_ref[pl.ds(my_i * m_per, m_per), :] = y

        @pl.when(jj > 0)
        def _():
            y_scratch[...] = y
            rdma = pltpu.make_async_remote_copy(
                src_ref=y_scratch,
                dst_ref=out_ref.at[pl.ds(my_i * m_per, m_per), :],
                send_sem=send_sem,
                recv_sem=recv_sem,
                device_id=(tgt,),
                device_id_type=pl.DeviceIdType.MESH,
            )
            rdma.start()
            rdma.wait_send()

        @pl.when(jj == N_DEV - 1)
        def _():
            for _ in range(N_DEV - 1):
                dummy = pltpu.make_async_remote_copy(
                    src_ref=y_scratch,
                    dst_ref=y_scratch,
                    send_sem=send_sem,
                    recv_sem=recv_sem,
                    device_id=(my_i,),
                    device_id_type=pl.DeviceIdType.MESH,
                )
                dummy.wait_recv()

    grid = (N_DEV,)
    return pl.pallas_call(
        body,
        grid=grid,
        in_specs=[
            pl.BlockSpec((m_per, k), lambda jj: (0, 0)),
            pl.BlockSpec(
                (k, n_per),
                lambda jj: (0, lax.rem(lax.axis_index("i") + jj, N_DEV)),
            ),
            pl.BlockSpec(memory_space=pltpu.SMEM),
            pl.BlockSpec(memory_space=pltpu.SMEM),
        ],
        out_specs=pl.BlockSpec((N_DEV * m_per, n_per), lambda jj: (0, 0)),
        out_shape=jax.ShapeDtypeStruct((N_DEV * m_per, n_per), jnp.float32),
        scratch_shapes=[
            pltpu.VMEM((m_per, n_per), jnp.float32),
            pltpu.SemaphoreType.DMA,
            pltpu.SemaphoreType.DMA,
        ],
        compiler_params=pltpu.CompilerParams(
            dimension_semantics=("arbitrary",),
        ),
    )(x, w_mat, scale_x, scale_w)


# device time: 111573 ns/iter; 1.5753x vs baseline; 1.5753x over previous
import jax
import jax.numpy as jnp
from jax import lax
from jax.experimental import pallas as pl
from jax.experimental.pallas import tpu as pltpu

N_DEV = 16


def kernel(x, w_mat, scale_x, scale_w):
    m_per, k = x.shape
    _, n = w_mat.shape
    n_per = n // N_DEV

    def body(x_ref, w_ref, sx_ref, sw_ref, out_ref, y_scratch, send_sems, recv_sem):
        jj = pl.program_id(0)
        my_i = lax.axis_index("i")
        tgt = lax.rem(my_i + jj, N_DEV)

        scale = sx_ref[0] * sw_ref[0]
        xb = x_ref[...].astype(jnp.bfloat16)
        wb = w_ref[...].astype(jnp.bfloat16)
        y = jnp.dot(xb, wb, preferred_element_type=jnp.float32) * scale

        @pl.when(jj == 0)
        def _():
            out_ref[pl.ds(my_i * m_per, m_per), :] = y

        @pl.when(jj > 0)
        def _():
            slot = jj - 1
            y_scratch[slot] = y
            rdma = pltpu.make_async_remote_copy(
                src_ref=y_scratch.at[slot],
                dst_ref=out_ref.at[pl.ds(my_i * m_per, m_per), :],
                send_sem=send_sems.at[slot],
                recv_sem=recv_sem,
                device_id=(tgt,),
                device_id_type=pl.DeviceIdType.MESH,
            )
            rdma.start()

        @pl.when(jj == N_DEV - 1)
        def _():
            for s in range(N_DEV - 1):
                dummy = pltpu.make_async_remote_copy(
                    src_ref=y_scratch.at[s],
                    dst_ref=y_scratch.at[s],
                    send_sem=send_sems.at[s],
                    recv_sem=recv_sem,
                    device_id=(my_i,),
                    device_id_type=pl.DeviceIdType.MESH,
                )
                dummy.wait_send()
                dummy.wait_recv()

    grid = (N_DEV,)
    return pl.pallas_call(
        body,
        grid=grid,
        in_specs=[
            pl.BlockSpec((m_per, k), lambda jj: (0, 0)),
            pl.BlockSpec(
                (k, n_per),
                lambda jj: (0, lax.rem(lax.axis_index("i") + jj, N_DEV)),
            ),
            pl.BlockSpec(memory_space=pltpu.SMEM),
            pl.BlockSpec(memory_space=pltpu.SMEM),
        ],
        out_specs=pl.BlockSpec((N_DEV * m_per, n_per), lambda jj: (0, 0)),
        out_shape=jax.ShapeDtypeStruct((N_DEV * m_per, n_per), jnp.float32),
        scratch_shapes=[
            pltpu.VMEM((N_DEV - 1, m_per, n_per), jnp.float32),
            pltpu.SemaphoreType.DMA((N_DEV - 1,)),
            pltpu.SemaphoreType.DMA,
        ],
        compiler_params=pltpu.CompilerParams(
            dimension_semantics=("arbitrary",),
        ),
    )(x, w_mat, scale_x, scale_w)


# device time: 48202 ns/iter; 3.6464x vs baseline; 2.3147x over previous
import jax
import jax.numpy as jnp
from jax import lax
from jax.experimental import pallas as pl
from jax.experimental.pallas import tpu as pltpu

N_DEV = 16


def kernel(x, w_mat, scale_x, scale_w):
    m_per, k = x.shape
    _, n = w_mat.shape
    n_per = n // N_DEV

    def body(x_ref, w_ref, sx_ref, sw_ref, out_ref, xq_scratch, y_scratch,
             send_sems, recv_sem):
        jj = pl.program_id(0)
        my_i = lax.axis_index("i")
        tgt = lax.rem(my_i + jj, N_DEV)

        @pl.when(jj == 0)
        def _():
            xq_scratch[...] = x_ref[...].astype(jnp.float8_e4m3fn)

        scale = sx_ref[0] * sw_ref[0]
        wq = w_ref[...].astype(jnp.float8_e5m2)
        y = jnp.dot(xq_scratch[...], wq, preferred_element_type=jnp.float32) * scale

        @pl.when(jj == 0)
        def _():
            out_ref[pl.ds(my_i * m_per, m_per), :] = y

        @pl.when(jj > 0)
        def _():
            slot = jj - 1
            y_scratch[slot] = y
            rdma = pltpu.make_async_remote_copy(
                src_ref=y_scratch.at[slot],
                dst_ref=out_ref.at[pl.ds(my_i * m_per, m_per), :],
                send_sem=send_sems.at[slot],
                recv_sem=recv_sem,
                device_id=(tgt,),
                device_id_type=pl.DeviceIdType.MESH,
            )
            rdma.start()

        @pl.when(jj == N_DEV - 1)
        def _():
            for s in range(N_DEV - 1):
                dummy = pltpu.make_async_remote_copy(
                    src_ref=y_scratch.at[s],
                    dst_ref=y_scratch.at[s],
                    send_sem=send_sems.at[s],
                    recv_sem=recv_sem,
                    device_id=(my_i,),
                    device_id_type=pl.DeviceIdType.MESH,
                )
                dummy.wait_send()
                dummy.wait_recv()

    grid = (N_DEV,)
    return pl.pallas_call(
        body,
        grid=grid,
        in_specs=[
            pl.BlockSpec((m_per, k), lambda jj: (0, 0)),
            pl.BlockSpec(
                (k, n_per),
                lambda jj: (0, lax.rem(lax.axis_index("i") + jj, N_DEV)),
            ),
            pl.BlockSpec(memory_space=pltpu.SMEM),
            pl.BlockSpec(memory_space=pltpu.SMEM),
        ],
        out_specs=pl.BlockSpec((N_DEV * m_per, n_per), lambda jj: (0, 0)),
        out_shape=jax.ShapeDtypeStruct((N_DEV * m_per, n_per), jnp.float32),
        scratch_shapes=[
            pltpu.VMEM((m_per, k), jnp.float8_e4m3fn),
            pltpu.VMEM((N_DEV - 1, m_per, n_per), jnp.float32),
            pltpu.SemaphoreType.DMA((N_DEV - 1,)),
            pltpu.SemaphoreType.DMA,
        ],
        compiler_params=pltpu.CompilerParams(
            dimension_semantics=("arbitrary",),
        ),
    )(x, w_mat, scale_x, scale_w)
